# baseline (speedup 1.0000x reference)
import jax
import jax.numpy as jnp
from jax import lax
from jax.experimental import pallas as pl
from jax.experimental.pallas import tpu as pltpu
from jax.experimental.pallas import tpu_sc as plsc

NC, NS = 2, 16
NW = NC * NS
K = 80


def _sc_conv(h, row_s, col_s, w_s):
    n, d = h.shape
    nsup, csub = row_s.shape[1], row_s.shape[2]
    na = -(-n // (NS * K)) * NS * K
    rpt = na // NS
    mesh = plsc.VectorSubcoreMesh(core_axis_name="c", subcore_axis_name="s")

    def body(h_hbm, row_hbm, col_hbm, w_hbm, out_hbm,
             row_v, col_v, w_v, rows_a, rows_b, rows_c,
             gsem_a, gsem_b, gsem_c, ssem_a, ssem_b, ssem_c, lsem,
             acc_sh):
        c = lax.axis_index("c")
        s = lax.axis_index("s")
        wid = s * NC + c
        bufs = [(rows_a, gsem_a, ssem_a),
                (rows_b, gsem_b, ssem_b),
                (rows_c, gsem_c, ssem_c)]

        def slab_load(q, p):
            pltpu.async_copy(row_hbm.at[wid, q], row_v.at[p], lsem)
            pltpu.async_copy(col_hbm.at[wid, q], col_v.at[p], lsem)
            pltpu.async_copy(w_hbm.at[wid, q], w_v.at[p], lsem)

        def slab_wait(q, p):
            pltpu.make_async_copy(row_hbm.at[wid, q], row_v.at[p],
                                  lsem).wait()
            pltpu.make_async_copy(col_hbm.at[wid, q], col_v.at[p],
                                  lsem).wait()
            pltpu.make_async_copy(w_hbm.at[wid, q], w_v.at[p], lsem).wait()

        slab_load(0, 0)
        slab_load(1, 1)

        def z1(i, _):
            rows_a[i // (d // 16), pl.ds((i % (d // 16)) * 16, 16)] = (
                jnp.zeros((16,), jnp.float32))
            return 0
        lax.fori_loop(0, K * (d // 16), z1, 0)

        def z2(r, _):
            pltpu.sync_copy(rows_a, acc_sh.at[pl.ds(s * rpt + r * K, K)])
            return 0
        lax.fori_loop(0, rpt // K, z2, 0)
        plsc.subcore_barrier()

        def scale(buf, p, g):
            def edge16(eb, _):
                wvec = w_v[p, g, pl.ds(eb * 16, 16)]
                for l in range(16):
                    ws = wvec[l]
                    for j in range(d // 16):
                        buf[eb * 16 + l, pl.ds(j * 16, 16)] = (
                            buf[eb * 16 + l, pl.ds(j * 16, 16)] * ws)
                return 0
            lax.fori_loop(0, K // 16, edge16, 0, unroll=True)

        def gather(p, g, b):
            pltpu.async_copy(h_hbm.at[row_v.at[p, g]], bufs[b][0],
                             bufs[b][1])

        def gwait(p, g, b):
            pltpu.make_async_copy(h_hbm.at[row_v.at[p, g]], bufs[b][0],
                                  bufs[b][1]).wait()

        def scat(p, g, b):
            pltpu.async_copy(bufs[b][0], acc_sh.at[col_v.at[p, g]],
                             bufs[b][2], add=True)

        def swait(b):
            pltpu.make_async_copy(bufs[b][0], acc_sh.at[col_v.at[0, 0]],
                                  bufs[b][2]).wait()

        nch = nsup * csub

        def gstep(g, b, prefetch=True):
            q = g // csub
            p = q % 2
            gl = g % csub
            gwait(p, gl, b)
            scale(bufs[b][0], p, gl)
            scat(p, gl, b)
            if prefetch:
                qn = jnp.minimum(q + 1, nsup - 1)

                @pl.when((gl == 1) & (q < nsup - 1))
                def _():
                    slab_load(q + 1, 1 - p)

                @pl.when(gl == 3)
                def _():
                    slab_wait(qn, 1 - p)

                nb = (b + 2) % 3
                swait(nb)
                g2 = g + 2
                gather((g2 // csub) % 2, g2 % csub, nb)

        slab_wait(0, 0)

        gather(0, 0, 0)
        gather(0, 1, 1)
        gwait(0, 0, 0)
        scale(rows_a, 0, 0)
        scat(0, 0, 0)
        gather(0, 2, 2)
        gwait(0, 1, 1)
        scale(rows_b, 0, 1)
        scat(0, 1, 1)
        swait(0)
        gather(0, 3, 0)
        gwait(0, 2, 2)
        scale(rows_c, 0, 2)
        scat(0, 2, 2)
        swait(1)
        gather(0, 4, 1)

        def chunk3(gg, _):
            g0 = gg * 3 + 3
            gstep(g0, 0)
            gstep(g0 + 1, 1)
            gstep(g0 + 2, 2)
            return 0
        lax.fori_loop(0, (nch - 5) // 3, chunk3, 0)

        gstep(nch - 2, (nch - 2) % 3, prefetch=False)
        gstep(nch - 1, (nch - 1) % 3, prefetch=False)

        swait((nch - 3) % 3)
        swait((nch - 2) % 3)
        swait((nch - 1) % 3)
        plsc.subcore_barrier()

        pltpu.sync_copy(acc_sh.at[pl.ds(s * rpt, rpt)],
                        out_hbm.at[c, pl.ds(s * rpt, rpt)])

    return pl.kernel(
        body,
        out_type=jax.ShapeDtypeStruct((NC, na, d), jnp.float32),
        mesh=mesh,
        scratch_types=[
            pltpu.VMEM((2, csub, K), jnp.int32),
            pltpu.VMEM((2, csub, K), jnp.int32),
            pltpu.VMEM((2, csub, K), jnp.float32),
            pltpu.VMEM((K, d), jnp.float32),
            pltpu.VMEM((K, d), jnp.float32),
            pltpu.VMEM((K, d), jnp.float32),
            pltpu.SemaphoreType.DMA,
            pltpu.SemaphoreType.DMA,
            pltpu.SemaphoreType.DMA,
            pltpu.SemaphoreType.DMA,
            pltpu.SemaphoreType.DMA,
            pltpu.SemaphoreType.DMA,
            pltpu.SemaphoreType.DMA,
            pltpu.VMEM_SHARED((na, d), jnp.float32),
        ],
    )(h, row_s, col_s, w_s)


def _mm(x, W):
    n, d = x.shape
    dout = W.shape[1]
    blk = 2000

    def body(x_ref, w_ref, o_ref):
        o_ref[...] = jnp.dot(x_ref[...], w_ref[...],
                             preferred_element_type=jnp.float32)

    return pl.pallas_call(
        body,
        grid=(n // blk,),
        in_specs=[pl.BlockSpec((blk, d), lambda i: (i, 0)),
                  pl.BlockSpec((d, dout), lambda i: (0, 0))],
        out_specs=pl.BlockSpec((blk, dout), lambda i: (i, 0)),
        out_shape=jax.ShapeDtypeStruct((n, dout), jnp.float32),
    )(x, W)


def _combine_mm(p0, p1, b, W):
    n, d = p0.shape
    dout = W.shape[1]
    blk = 2000

    def body(p0_ref, p1_ref, b_ref, w_ref, o_ref):
        t = jnp.maximum(p0_ref[...] + p1_ref[...] + b_ref[...], 0.0)
        o_ref[...] = jnp.dot(t, w_ref[...],
                             preferred_element_type=jnp.float32)

    return pl.pallas_call(
        body,
        grid=(n // blk,),
        in_specs=[pl.BlockSpec((blk, d), lambda i: (i, 0)),
                  pl.BlockSpec((blk, d), lambda i: (i, 0)),
                  pl.BlockSpec((1, d), lambda i: (0, 0)),
                  pl.BlockSpec((d, dout), lambda i: (0, 0))],
        out_specs=pl.BlockSpec((blk, dout), lambda i: (i, 0)),
        out_shape=jax.ShapeDtypeStruct((n, dout), jnp.float32),
    )(p0, p1, b, W)


def _head(p0, p1, b2, f1W, f1b, f2W, f2b):
    n, d = p0.shape
    dh = f1W.shape[1]
    dout = f2W.shape[1]
    blk = 2000

    def body(p0_ref, p1_ref, b2_ref, f1w_ref, f1b_ref, f2w_ref, f2b_ref,
             o_ref):
        t = p0_ref[...] + p1_ref[...] + b2_ref[...]
        hh = jnp.maximum(
            jnp.dot(t, f1w_ref[...], preferred_element_type=jnp.float32)
            + f1b_ref[...], 0.0)
        o_ref[...] = jnp.dot(hh, f2w_ref[...],
                             preferred_element_type=jnp.float32) + f2b_ref[...]

    return pl.pallas_call(
        body,
        grid=(n // blk,),
        in_specs=[pl.BlockSpec((blk, d), lambda i: (i, 0)),
                  pl.BlockSpec((blk, d), lambda i: (i, 0)),
                  pl.BlockSpec((1, d), lambda i: (0, 0)),
                  pl.BlockSpec((d, dh), lambda i: (0, 0)),
                  pl.BlockSpec((1, dh), lambda i: (0, 0)),
                  pl.BlockSpec((dh, dout), lambda i: (0, 0)),
                  pl.BlockSpec((1, dout), lambda i: (0, 0))],
        out_specs=pl.BlockSpec((blk, dout), lambda i: (i, 0)),
        out_shape=jax.ShapeDtypeStruct((n, dout), jnp.float32),
    )(p0, p1, b2, f1W, f1b, f2W, f2b)


def kernel(x, edge_index, edge_attr, W1, b1, W2, b2, fc1_W, fc1_b, fc2_W,
           fc2_b):
    n, d = x.shape
    e = edge_attr.shape[0]
    csub = 5
    nsup = e // (NW * K * csub)
    row_s = edge_index[0].reshape(NW, nsup, csub, K)
    col_s = edge_index[1].reshape(NW, nsup, csub, K)
    w_s = edge_attr.reshape(NW, nsup, csub, K)

    h1p = _mm(x, W1)
    p1 = _sc_conv(h1p, row_s, col_s, w_s)
    h2p = _combine_mm(p1[0, :n], p1[1, :n], b1.reshape(1, -1), W2)
    p2 = _sc_conv(h2p, row_s, col_s, w_s)
    return _head(p2[0, :n], p2[1, :n], b2.reshape(1, -1), fc1_W,
                 fc1_b.reshape(1, -1), fc2_W, fc2_b.reshape(1, -1))


# speedup vs baseline: 1.3320x; 1.3320x over previous
import jax
import jax.numpy as jnp
from jax import lax
from jax.experimental import pallas as pl
from jax.experimental.pallas import tpu as pltpu
from jax.experimental.pallas import tpu_sc as plsc

NC, NS = 2, 16
NW = NC * NS
K = 80


def _sc_conv(h, row_s, col_s, w_s):
    n, d = h.shape
    nsup, csub = row_s.shape[1], row_s.shape[2]
    na = -(-n // (NS * K)) * NS * K
    rpt = na // NS
    mesh = plsc.VectorSubcoreMesh(core_axis_name="c", subcore_axis_name="s")

    def body(h_hbm, row_hbm, col_hbm, w_hbm, out_hbm,
             row_v, col_v, w_v, rows_a, rows_b, rows_c,
             gsem_a, gsem_b, gsem_c, ssem_a, ssem_b, ssem_c, lsem,
             acc_sh):
        c = lax.axis_index("c")
        s = lax.axis_index("s")
        wid = s * NC + c
        bufs = [(rows_a, gsem_a, ssem_a),
                (rows_b, gsem_b, ssem_b),
                (rows_c, gsem_c, ssem_c)]

        def slab_load(q, p):
            pltpu.async_copy(row_hbm.at[wid, q], row_v.at[p], lsem)
            pltpu.async_copy(col_hbm.at[wid, q], col_v.at[p], lsem)
            pltpu.async_copy(w_hbm.at[wid, q], w_v.at[p], lsem)

        def slab_wait(q, p):
            pltpu.make_async_copy(row_hbm.at[wid, q], row_v.at[p],
                                  lsem).wait()
            pltpu.make_async_copy(col_hbm.at[wid, q], col_v.at[p],
                                  lsem).wait()
            pltpu.make_async_copy(w_hbm.at[wid, q], w_v.at[p], lsem).wait()

        slab_load(0, 0)
        slab_load(1, 1)

        def z1(i, _):
            rows_a[i // (d // 16), pl.ds((i % (d // 16)) * 16, 16)] = (
                jnp.zeros((16,), jnp.float32))
            return 0
        lax.fori_loop(0, K * (d // 16), z1, 0)

        def z2(r, _):
            pltpu.sync_copy(rows_a, acc_sh.at[pl.ds(s * rpt + r * K, K)])
            return 0
        lax.fori_loop(0, rpt // K, z2, 0)
        plsc.subcore_barrier()

        def scale(buf, p, g):
            def edge16(eb, _):
                wvec = w_v[p, g, pl.ds(eb * 16, 16)]
                for l in range(16):
                    ws = wvec[l]
                    for j in range(d // 16):
                        buf[eb * 16 + l, pl.ds(j * 16, 16)] = (
                            buf[eb * 16 + l, pl.ds(j * 16, 16)] * ws)
                return 0
            lax.fori_loop(0, K // 16, edge16, 0)

        def gather(p, g, b):
            pltpu.async_copy(h_hbm.at[row_v.at[p, g]], bufs[b][0],
                             bufs[b][1])

        def gwait(p, g, b):
            pltpu.make_async_copy(h_hbm.at[row_v.at[p, g]], bufs[b][0],
                                  bufs[b][1]).wait()

        def scat(p, g, b):
            pltpu.async_copy(bufs[b][0], acc_sh.at[col_v.at[p, g]],
                             bufs[b][2], add=True)

        def swait(b):
            pltpu.make_async_copy(bufs[b][0], acc_sh.at[col_v.at[0, 0]],
                                  bufs[b][2]).wait()

        nch = nsup * csub

        def gstep(g, b, prefetch=True):
            q = g // csub
            p = q % 2
            gl = g % csub
            gwait(p, gl, b)
            if prefetch:
                qn = jnp.minimum(q + 1, nsup - 1)

                @pl.when((gl == 1) & (q < nsup - 1))
                def _():
                    slab_load(q + 1, 1 - p)

                @pl.when(gl == 3)
                def _():
                    slab_wait(qn, 1 - p)

                nb = (b + 2) % 3
                swait(nb)
                g2 = g + 2
                gather((g2 // csub) % 2, g2 % csub, nb)
            scale(bufs[b][0], p, gl)
            scat(p, gl, b)

        slab_wait(0, 0)

        gather(0, 0, 0)
        gather(0, 1, 1)
        gwait(0, 0, 0)
        scale(rows_a, 0, 0)
        scat(0, 0, 0)
        gather(0, 2, 2)
        gwait(0, 1, 1)
        scale(rows_b, 0, 1)
        scat(0, 1, 1)
        swait(0)
        gather(0, 3, 0)
        gwait(0, 2, 2)
        scale(rows_c, 0, 2)
        scat(0, 2, 2)
        swait(1)
        gather(0, 4, 1)

        def chunk3(gg, _):
            g0 = gg * 3 + 3
            gstep(g0, 0)
            gstep(g0 + 1, 1)
            gstep(g0 + 2, 2)
            return 0
        lax.fori_loop(0, (nch - 5) // 3, chunk3, 0)

        gstep(nch - 2, (nch - 2) % 3, prefetch=False)
        gstep(nch - 1, (nch - 1) % 3, prefetch=False)

        swait((nch - 3) % 3)
        swait((nch - 2) % 3)
        swait((nch - 1) % 3)
        plsc.subcore_barrier()

        pltpu.sync_copy(acc_sh.at[pl.ds(s * rpt, rpt)],
                        out_hbm.at[c, pl.ds(s * rpt, rpt)])

    return pl.kernel(
        body,
        out_type=jax.ShapeDtypeStruct((NC, na, d), jnp.float32),
        mesh=mesh,
        scratch_types=[
            pltpu.VMEM((2, csub, K), jnp.int32),
            pltpu.VMEM((2, csub, K), jnp.int32),
            pltpu.VMEM((2, csub, K), jnp.float32),
            pltpu.VMEM((K, d), jnp.float32),
            pltpu.VMEM((K, d), jnp.float32),
            pltpu.VMEM((K, d), jnp.float32),
            pltpu.SemaphoreType.DMA,
            pltpu.SemaphoreType.DMA,
            pltpu.SemaphoreType.DMA,
            pltpu.SemaphoreType.DMA,
            pltpu.SemaphoreType.DMA,
            pltpu.SemaphoreType.DMA,
            pltpu.SemaphoreType.DMA,
            pltpu.VMEM_SHARED((na, d), jnp.float32),
        ],
    )(h, row_s, col_s, w_s)


def _mm(x, W):
    n, d = x.shape
    dout = W.shape[1]
    blk = 2000

    def body(x_ref, w_ref, o_ref):
        o_ref[...] = jnp.dot(x_ref[...], w_ref[...],
                             preferred_element_type=jnp.float32)

    return pl.pallas_call(
        body,
        grid=(n // blk,),
        in_specs=[pl.BlockSpec((blk, d), lambda i: (i, 0)),
                  pl.BlockSpec((d, dout), lambda i: (0, 0))],
        out_specs=pl.BlockSpec((blk, dout), lambda i: (i, 0)),
        out_shape=jax.ShapeDtypeStruct((n, dout), jnp.float32),
    )(x, W)


def _combine_mm(p0, p1, b, W):
    n, d = p0.shape
    dout = W.shape[1]
    blk = 2000

    def body(p0_ref, p1_ref, b_ref, w_ref, o_ref):
        t = jnp.maximum(p0_ref[...] + p1_ref[...] + b_ref[...], 0.0)
        o_ref[...] = jnp.dot(t, w_ref[...],
                             preferred_element_type=jnp.float32)

    return pl.pallas_call(
        body,
        grid=(n // blk,),
        in_specs=[pl.BlockSpec((blk, d), lambda i: (i, 0)),
                  pl.BlockSpec((blk, d), lambda i: (i, 0)),
                  pl.BlockSpec((1, d), lambda i: (0, 0)),
                  pl.BlockSpec((d, dout), lambda i: (0, 0))],
        out_specs=pl.BlockSpec((blk, dout), lambda i: (i, 0)),
        out_shape=jax.ShapeDtypeStruct((n, dout), jnp.float32),
    )(p0, p1, b, W)


def _head(p0, p1, b2, f1W, f1b, f2W, f2b):
    n, d = p0.shape
    dh = f1W.shape[1]
    dout = f2W.shape[1]
    blk = 2000

    def body(p0_ref, p1_ref, b2_ref, f1w_ref, f1b_ref, f2w_ref, f2b_ref,
             o_ref):
        t = p0_ref[...] + p1_ref[...] + b2_ref[...]
        hh = jnp.maximum(
            jnp.dot(t, f1w_ref[...], preferred_element_type=jnp.float32)
            + f1b_ref[...], 0.0)
        o_ref[...] = jnp.dot(hh, f2w_ref[...],
                             preferred_element_type=jnp.float32) + f2b_ref[...]

    return pl.pallas_call(
        body,
        grid=(n // blk,),
        in_specs=[pl.BlockSpec((blk, d), lambda i: (i, 0)),
                  pl.BlockSpec((blk, d), lambda i: (i, 0)),
                  pl.BlockSpec((1, d), lambda i: (0, 0)),
                  pl.BlockSpec((d, dh), lambda i: (0, 0)),
                  pl.BlockSpec((1, dh), lambda i: (0, 0)),
                  pl.BlockSpec((dh, dout), lambda i: (0, 0)),
                  pl.BlockSpec((1, dout), lambda i: (0, 0))],
        out_specs=pl.BlockSpec((blk, dout), lambda i: (i, 0)),
        out_shape=jax.ShapeDtypeStruct((n, dout), jnp.float32),
    )(p0, p1, b2, f1W, f1b, f2W, f2b)


def kernel(x, edge_index, edge_attr, W1, b1, W2, b2, fc1_W, fc1_b, fc2_W,
           fc2_b):
    n, d = x.shape
    e = edge_attr.shape[0]
    csub = 5
    nsup = e // (NW * K * csub)
    row_s = edge_index[0].reshape(NW, nsup, csub, K)
    col_s = edge_index[1].reshape(NW, nsup, csub, K)
    w_s = edge_attr.reshape(NW, nsup, csub, K)

    h1p = _mm(x, W1)
    p1 = _sc_conv(h1p, row_s, col_s, w_s)
    h2p = _combine_mm(p1[0, :n], p1[1, :n], b1.reshape(1, -1), W2)
    p2 = _sc_conv(h2p, row_s, col_s, w_s)
    return _head(p2[0, :n], p2[1, :n], b2.reshape(1, -1), fc1_W,
                 fc1_b.reshape(1, -1), fc2_W, fc2_b.reshape(1, -1))
